# baseline (speedup 1.0000x reference)
import functools

import jax
import jax.numpy as jnp
from jax.experimental import pallas as pl
from jax.experimental.pallas import tpu as pltpu

_IMG = 224
_PATCH = 32
_P = 7
_NP = _P * _P
_PATCH_DIM = 3 * _PATCH * _PATCH
_YX = _PATCH * _PATCH
_FEAT = 2048
_NCLS_PAD = 1024

_VMEM_LIMIT = 60 * 1024 * 1024


def _patchify_kernel(x_ref, o_ref):
    v = x_ref[:, 0, 0].astype(jnp.bfloat16)
    pieces = [v[:, :, pw * _PATCH:(pw + 1) * _PATCH][:, None]
              for pw in range(_P)]
    o_ref[:, 0, 0] = jnp.concatenate(pieces, axis=1)


def _patchify(x, *, blk):
    n = x.shape[0]
    x5 = x.reshape(n, 3, _P, _PATCH, _IMG)
    out = pl.pallas_call(
        _patchify_kernel,
        out_shape=jax.ShapeDtypeStruct((n, _P, 3, _P, _PATCH, _PATCH),
                                       jnp.bfloat16),
        grid=(n // blk, 3, _P),
        in_specs=[pl.BlockSpec((blk, 1, 1, _PATCH, _IMG),
                               lambda i, c, ph: (i, c, ph, 0, 0))],
        out_specs=pl.BlockSpec((blk, 1, 1, _P, _PATCH, _PATCH),
                               lambda i, c, ph: (i, ph, c, 0, 0, 0)),
        compiler_params=pltpu.CompilerParams(
            dimension_semantics=("parallel", "arbitrary", "arbitrary"),
            vmem_limit_bytes=_VMEM_LIMIT,
        ),
    )(x5)
    return out.reshape(n * _P, 3 * _P * _YX)


def _fused_kernel(a0_ref, a1_ref, a2_ref, tgt_ref, we_ref, be_ref, wf_ref,
                  bf_ref, loss_ref, s_ref, *, imgs):
    rows = imgs * _P
    pw = pl.program_id(1)

    a = jnp.concatenate([a0_ref[...], a1_ref[...], a2_ref[...]], axis=-1)
    h = jnp.dot(a, we_ref[...], preferred_element_type=jnp.float32)
    h = jnp.maximum(h + be_ref[...], 0.0)

    @pl.when(pw == 0)
    def _init():
        s_ref[...] = h

    @pl.when(pw > 0)
    def _acc():
        s_ref[...] = s_ref[...] + h

    @pl.when(pw == _P - 1)
    def _epilogue():
        row = jax.lax.broadcasted_iota(jnp.int32, (imgs, rows), 0)
        col = jax.lax.broadcasted_iota(jnp.int32, (imgs, rows), 1)
        lo = row * _P
        pool = jnp.where((col >= lo) & (col < lo + _P),
                         jnp.float32(1.0 / _NP), jnp.float32(0.0))
        pooled = jnp.dot(pool, s_ref[...], preferred_element_type=jnp.float32)

        feats = pooled.astype(jnp.bfloat16)
        logits = jnp.dot(feats, wf_ref[...],
                         preferred_element_type=jnp.float32) + bf_ref[...]
        m = jnp.max(logits, axis=-1, keepdims=True)
        lse = m + jnp.log(jnp.sum(jnp.exp(logits - m), axis=-1, keepdims=True))
        cls_ids = jax.lax.broadcasted_iota(jnp.int32, logits.shape, 1)
        tgt_logit = jnp.sum(jnp.where(cls_ids == tgt_ref[...], logits, 0.0),
                            axis=-1, keepdims=True)
        loss_ref[...] = lse - tgt_logit


def _forward(g, target, w_embed, b_embed, w_fc, b_fc, *, imgs):
    n = target.shape[0]
    nb = n // imgs
    rows = imgs * _P
    body = functools.partial(_fused_kernel, imgs=imgs)

    def _a_spec(c):
        return pl.BlockSpec((rows, _YX), lambda i, pw, c=c: (i, c * _P + pw))

    return pl.pallas_call(
        body,
        out_shape=jax.ShapeDtypeStruct((n, 1), jnp.float32),
        grid=(nb, _P),
        in_specs=[
            _a_spec(0),
            _a_spec(1),
            _a_spec(2),
            pl.BlockSpec((imgs, 1), lambda i, pw: (i, 0)),
            pl.BlockSpec((_PATCH_DIM, _FEAT), lambda i, pw: (0, 0)),
            pl.BlockSpec((1, _FEAT), lambda i, pw: (0, 0)),
            pl.BlockSpec((_FEAT, _NCLS_PAD), lambda i, pw: (0, 0)),
            pl.BlockSpec((1, _NCLS_PAD), lambda i, pw: (0, 0)),
        ],
        out_specs=pl.BlockSpec((imgs, 1), lambda i, pw: (i, 0)),
        scratch_shapes=[pltpu.VMEM((rows, _FEAT), jnp.float32)],
        compiler_params=pltpu.CompilerParams(
            dimension_semantics=("parallel", "arbitrary"),
            vmem_limit_bytes=_VMEM_LIMIT,
        ),
    )(g, g, g, target.reshape(n, 1), w_embed, b_embed, w_fc, b_fc)


def _pick(n, cands):
    for c in cands:
        if n % c == 0:
            return c
    return None


@jax.jit
def kernel(x, target, w_embed, b_embed, w_fc, b_fc):
    n = x.shape[0]
    imgs = _pick(n, (128, 64, 32, 16, 8))
    if imgs is None:
        imgs = min(n, 8)
    n_eff = (n + imgs - 1) // imgs * imgs
    target = target.astype(jnp.int32)
    if n_eff != n:
        x = jnp.pad(x, ((0, n_eff - n), (0, 0), (0, 0), (0, 0)))
        target = jnp.pad(target, (0, n_eff - n))
    blk = _pick(n_eff, (64, 32, 16, 8, 4, 2, 1))
    g = _patchify(x, blk=blk)
    losses = _forward(g, target, w_embed, b_embed, w_fc, b_fc, imgs=imgs)
    return jnp.mean(losses[:n, 0])


# speedup vs baseline: 1.0146x; 1.0146x over previous
import functools

import jax
import jax.numpy as jnp
from jax.experimental import pallas as pl
from jax.experimental.pallas import tpu as pltpu

_IMG = 224
_PATCH = 32
_P = 7
_NP = _P * _P
_PATCH_DIM = 3 * _PATCH * _PATCH
_YX = _PATCH * _PATCH
_FEAT = 2048
_NCLS_PAD = 1024

_VMEM_LIMIT = 60 * 1024 * 1024


def _fused_kernel(a0_ref, a1_ref, a2_ref, tgt_ref, we_ref, be_ref, wf_ref,
                  bf_ref, loss_ref, s_ref, *, imgs):
    rows = imgs * _P
    pw = pl.program_id(1)

    a = jnp.concatenate([a0_ref[...], a1_ref[...], a2_ref[...]], axis=-1)
    h = jnp.dot(a, we_ref[...], preferred_element_type=jnp.float32)
    h = jnp.maximum(h + be_ref[...], 0.0)

    @pl.when(pw == 0)
    def _init():
        s_ref[...] = h

    @pl.when(pw > 0)
    def _acc():
        s_ref[...] = s_ref[...] + h

    @pl.when(pw == _P - 1)
    def _epilogue():
        row = jax.lax.broadcasted_iota(jnp.int32, (imgs, rows), 0)
        col = jax.lax.broadcasted_iota(jnp.int32, (imgs, rows), 1)
        lo = row * _P
        pool = jnp.where((col >= lo) & (col < lo + _P),
                         jnp.float32(1.0 / _NP), jnp.float32(0.0))
        pooled = jnp.dot(pool, s_ref[...], preferred_element_type=jnp.float32)

        feats = pooled.astype(jnp.bfloat16)
        logits = jnp.dot(feats, wf_ref[...],
                         preferred_element_type=jnp.float32) + bf_ref[...]
        m = jnp.max(logits, axis=-1, keepdims=True)
        lse = m + jnp.log(jnp.sum(jnp.exp(logits - m), axis=-1, keepdims=True))
        cls_ids = jax.lax.broadcasted_iota(jnp.int32, logits.shape, 1)
        tgt_logit = jnp.sum(jnp.where(cls_ids == tgt_ref[...], logits, 0.0),
                            axis=-1, keepdims=True)
        loss_ref[...] = lse - tgt_logit


def _forward(g, target, w_embed, b_embed, w_fc, b_fc, *, imgs):
    n = target.shape[0]
    nb = n // imgs
    rows = imgs * _P
    body = functools.partial(_fused_kernel, imgs=imgs)

    def _a_spec(c):
        return pl.BlockSpec((rows, _YX), lambda i, pw, c=c: (i, c * _P + pw))

    return pl.pallas_call(
        body,
        out_shape=jax.ShapeDtypeStruct((n, 1), jnp.float32),
        grid=(nb, _P),
        in_specs=[
            _a_spec(0),
            _a_spec(1),
            _a_spec(2),
            pl.BlockSpec((imgs, 1), lambda i, pw: (i, 0)),
            pl.BlockSpec((_PATCH_DIM, _FEAT), lambda i, pw: (0, 0)),
            pl.BlockSpec((1, _FEAT), lambda i, pw: (0, 0)),
            pl.BlockSpec((_FEAT, _NCLS_PAD), lambda i, pw: (0, 0)),
            pl.BlockSpec((1, _NCLS_PAD), lambda i, pw: (0, 0)),
        ],
        out_specs=pl.BlockSpec((imgs, 1), lambda i, pw: (i, 0)),
        scratch_shapes=[pltpu.VMEM((rows, _FEAT), jnp.float32)],
        compiler_params=pltpu.CompilerParams(
            dimension_semantics=("parallel", "arbitrary"),
            vmem_limit_bytes=_VMEM_LIMIT,
        ),
    )(g, g, g, target.reshape(n, 1), w_embed, b_embed, w_fc, b_fc)


def _patches(x):
    n = x.shape[0]
    xt = x.reshape(n, 3, _P, _PATCH, _P, _PATCH).transpose(0, 2, 1, 4, 3, 5)
    return xt.reshape(n * _P, 3 * _P * _YX)


@jax.jit
def kernel(x, target, w_embed, b_embed, w_fc, b_fc):
    n = x.shape[0]
    x = x.astype(jnp.bfloat16)
    target = target.astype(jnp.int32)

    if n % 128 == 0:
        h = n // 2
        parts = []
        for sl in (slice(0, h), slice(h, n)):
            g = _patches(x[sl])
            parts.append(_forward(g, target[sl], w_embed, b_embed, w_fc,
                                  b_fc, imgs=64))
        losses = jnp.concatenate(parts, axis=0)
        return jnp.mean(losses[:, 0])

    imgs = 8 if n % 8 == 0 else min(n, 8)
    n_eff = (n + imgs - 1) // imgs * imgs
    if n_eff != n:
        x = jnp.pad(x, ((0, n_eff - n), (0, 0), (0, 0), (0, 0)))
        target = jnp.pad(target, (0, n_eff - n))
    losses = _forward(_patches(x), target, w_embed, b_embed, w_fc, b_fc,
                      imgs=imgs)
    return jnp.mean(losses[:n, 0])


# speedup vs baseline: 1.1563x; 1.1397x over previous
import functools

import jax
import jax.numpy as jnp
from jax.experimental import pallas as pl
from jax.experimental.pallas import tpu as pltpu

_IMG = 224
_PATCH = 32
_P = 7
_NP = _P * _P
_PATCH_DIM = 3 * _PATCH * _PATCH
_YX = _PATCH * _PATCH
_FEAT = 2048
_NCLS_PAD = 1024

_VMEM_LIMIT = 60 * 1024 * 1024


def _fused_kernel(a0_ref, a1_ref, a2_ref, tgt_ref, we_ref, be_ref, wf_ref,
                  bf_ref, loss_ref, s_ref, *, imgs):
    rows = imgs * _P
    pw = pl.program_id(1)

    a = jnp.concatenate([a0_ref[...], a1_ref[...], a2_ref[...]], axis=-1)
    h = jnp.dot(a, we_ref[...], preferred_element_type=jnp.float32)
    h = jnp.maximum(h + be_ref[...], 0.0)

    @pl.when(pw == 0)
    def _init():
        s_ref[...] = h

    @pl.when(pw > 0)
    def _acc():
        s_ref[...] = s_ref[...] + h

    @pl.when(pw == _P - 1)
    def _epilogue():
        row = jax.lax.broadcasted_iota(jnp.int32, (imgs, rows), 0)
        col = jax.lax.broadcasted_iota(jnp.int32, (imgs, rows), 1)
        lo = row * _P
        pool = jnp.where((col >= lo) & (col < lo + _P),
                         jnp.float32(1.0 / _NP), jnp.float32(0.0))
        pooled = jnp.dot(pool, s_ref[...], preferred_element_type=jnp.float32)

        feats = pooled.astype(jnp.bfloat16)
        logits = jnp.dot(feats, wf_ref[...],
                         preferred_element_type=jnp.float32) + bf_ref[...]
        m = jnp.max(logits, axis=-1, keepdims=True)
        lse = m + jnp.log(jnp.sum(jnp.exp(logits - m), axis=-1, keepdims=True))
        cls_ids = jax.lax.broadcasted_iota(jnp.int32, logits.shape, 1)
        tgt_logit = jnp.sum(jnp.where(cls_ids == tgt_ref[...], logits, 0.0),
                            axis=-1, keepdims=True)
        loss_ref[...] = lse - tgt_logit


def _forward(g, target, w_embed, b_embed, w_fc, b_fc, *, imgs):
    n = target.shape[0]
    nb = n // imgs
    rows = imgs * _P
    body = functools.partial(_fused_kernel, imgs=imgs)

    def _a_spec(c):
        return pl.BlockSpec((rows, _YX), lambda i, pw, c=c: (i, c * _P + pw))

    return pl.pallas_call(
        body,
        out_shape=jax.ShapeDtypeStruct((n, 1), jnp.float32),
        grid=(nb, _P),
        in_specs=[
            _a_spec(0),
            _a_spec(1),
            _a_spec(2),
            pl.BlockSpec((imgs, 1), lambda i, pw: (i, 0)),
            pl.BlockSpec((_PATCH_DIM, _FEAT), lambda i, pw: (0, 0)),
            pl.BlockSpec((1, _FEAT), lambda i, pw: (0, 0)),
            pl.BlockSpec((_FEAT, _NCLS_PAD), lambda i, pw: (0, 0)),
            pl.BlockSpec((1, _NCLS_PAD), lambda i, pw: (0, 0)),
        ],
        out_specs=pl.BlockSpec((imgs, 1), lambda i, pw: (i, 0)),
        scratch_shapes=[pltpu.VMEM((rows, _FEAT), jnp.float32)],
        compiler_params=pltpu.CompilerParams(
            dimension_semantics=("parallel", "arbitrary"),
            vmem_limit_bytes=_VMEM_LIMIT,
        ),
    )(g, g, g, target.reshape(n, 1), w_embed, b_embed, w_fc, b_fc)


@jax.jit
def kernel(x, target, w_embed, b_embed, w_fc, b_fc):
    n = x.shape[0]
    x = x.astype(jnp.bfloat16)
    xt = x.reshape(n, 3, _P, _PATCH, _P, _PATCH).transpose(0, 2, 1, 4, 3, 5)
    g = xt.reshape(n * _P, 3 * _P * _YX)

    if n % 128 == 0:
        imgs = 128
    elif n % 8 == 0:
        imgs = 8
    else:
        imgs = min(n, 8)
    n_eff = (n + imgs - 1) // imgs * imgs
    target = target.astype(jnp.int32)
    if n_eff != n:
        g = jnp.pad(g, ((0, (n_eff - n) * _P), (0, 0)))
        target = jnp.pad(target, (0, n_eff - n))
    losses = _forward(g, target, w_embed, b_embed, w_fc, b_fc, imgs=imgs)
    return jnp.mean(losses[:n, 0])


# speedup vs baseline: 1.5090x; 1.3050x over previous
import functools

import jax
import jax.numpy as jnp
from jax.experimental import pallas as pl
from jax.experimental.pallas import tpu as pltpu

_IMG = 224
_PATCH = 32
_P = 7
_NP = _P * _P
_PATCH_DIM = 3 * _PATCH * _PATCH
_YX = _PATCH * _PATCH
_FEAT = 2048
_NCLS_PAD = 1024

_VMEM_LIMIT = 60 * 1024 * 1024


def _fused_kernel(a0_ref, a1_ref, a2_ref, tgt_ref, we_ref, be_ref, wf_ref,
                  bf_ref, loss_ref, s_ref, *, imgs, ph_major):
    rows = imgs * _P
    pw = pl.program_id(1)

    a = jnp.concatenate([a0_ref[...], a1_ref[...], a2_ref[...]], axis=-1)
    h = jnp.dot(a, we_ref[...], preferred_element_type=jnp.float32)
    h = jnp.maximum(h + be_ref[...], 0.0)

    @pl.when(pw == 0)
    def _init():
        s_ref[...] = h

    @pl.when(pw > 0)
    def _acc():
        s_ref[...] = s_ref[...] + h

    @pl.when(pw == _P - 1)
    def _epilogue():
        row = jax.lax.broadcasted_iota(jnp.int32, (imgs, rows), 0)
        col = jax.lax.broadcasted_iota(jnp.int32, (imgs, rows), 1)
        if ph_major:
            sel = col % imgs == row
        else:
            lo = row * _P
            sel = (col >= lo) & (col < lo + _P)
        pool = jnp.where(sel, jnp.float32(1.0 / _NP), jnp.float32(0.0))
        pooled = jnp.dot(pool, s_ref[...], preferred_element_type=jnp.float32)

        feats = pooled.astype(jnp.bfloat16)
        logits = jnp.dot(feats, wf_ref[...],
                         preferred_element_type=jnp.float32) + bf_ref[...]
        m = jnp.max(logits, axis=-1, keepdims=True)
        lse = m + jnp.log(jnp.sum(jnp.exp(logits - m), axis=-1, keepdims=True))
        cls_ids = jax.lax.broadcasted_iota(jnp.int32, logits.shape, 1)
        tgt_logit = jnp.sum(jnp.where(cls_ids == tgt_ref[...], logits, 0.0),
                            axis=-1, keepdims=True)
        loss_ref[...] = lse - tgt_logit


def _forward(g, target, w_embed, b_embed, w_fc, b_fc, *, imgs, ph_major):
    n = target.shape[0]
    nb = n // imgs
    rows = imgs * _P
    body = functools.partial(_fused_kernel, imgs=imgs, ph_major=ph_major)

    def _a_spec(c):
        return pl.BlockSpec((rows, _YX), lambda i, pw, c=c: (i, c * _P + pw))

    return pl.pallas_call(
        body,
        out_shape=jax.ShapeDtypeStruct((n, 1), jnp.float32),
        grid=(nb, _P),
        in_specs=[
            _a_spec(0),
            _a_spec(1),
            _a_spec(2),
            pl.BlockSpec((imgs, 1), lambda i, pw: (i, 0)),
            pl.BlockSpec((_PATCH_DIM, _FEAT), lambda i, pw: (0, 0)),
            pl.BlockSpec((1, _FEAT), lambda i, pw: (0, 0)),
            pl.BlockSpec((_FEAT, _NCLS_PAD), lambda i, pw: (0, 0)),
            pl.BlockSpec((1, _NCLS_PAD), lambda i, pw: (0, 0)),
        ],
        out_specs=pl.BlockSpec((imgs, 1), lambda i, pw: (i, 0)),
        scratch_shapes=[pltpu.VMEM((rows, _FEAT), jnp.float32)],
        compiler_params=pltpu.CompilerParams(
            dimension_semantics=("parallel", "arbitrary"),
            vmem_limit_bytes=_VMEM_LIMIT,
        ),
    )(g, g, g, target.reshape(n, 1), w_embed, b_embed, w_fc, b_fc)


@jax.jit
def kernel(x, target, w_embed, b_embed, w_fc, b_fc):
    n = x.shape[0]
    x = x.astype(jnp.bfloat16)
    x6 = x.reshape(n, 3, _P, _PATCH, _P, _PATCH)

    if n % 128 == 0:
        imgs = 128
    elif n % 8 == 0:
        imgs = 8
    else:
        imgs = min(n, 8)
    ph_major = imgs == n
    perm = (2, 0, 1, 4, 3, 5) if ph_major else (0, 2, 1, 4, 3, 5)
    g = x6.transpose(perm).reshape(n * _P, 3 * _P * _YX)

    n_eff = (n + imgs - 1) // imgs * imgs
    target = target.astype(jnp.int32)
    if n_eff != n:
        g = jnp.pad(g, ((0, (n_eff - n) * _P), (0, 0)))
        target = jnp.pad(target, (0, n_eff - n))
    losses = _forward(g, target, w_embed, b_embed, w_fc, b_fc, imgs=imgs,
                      ph_major=ph_major)
    return jnp.mean(losses[:n, 0])
